# Initial kernel scaffold; baseline (speedup 1.0000x reference)
import jax
import jax.numpy as jnp
from jax.experimental import pallas as pl


def kernel(x, edge_index, batch, W1n, W1s, b1, W2n, W2s, b2, W3n, W3s, b3, Wl, bl):
    raise NotImplementedError("write your pallas kernel here")


# speedup vs baseline: 6.0470x; 6.0470x over previous
import functools

import jax
import jax.numpy as jnp
from jax import lax
from jax.experimental import pallas as pl
from jax.experimental.pallas import tpu as pltpu
from jax.experimental.pallas import tpu_sc as plsc

N = 10000
E = 320000
H = 128
G = 64

NC = 2
NS = 16
NW = NC * NS
K = 128
NBLK = E // K
BASE_ITERS = NBLK // NW
EXTRA = NBLK - BASE_ITERS * NW
RPS = 624
TAIL = N - RPS * NS


def _sc_agg_body(feat, src, dst, part_out, src_v, dst_v, rows_v, acc, sem):
    c = lax.axis_index("c")
    s = lax.axis_index("s")
    wid = s * NC + c

    def zrow(r, _):
        for cc in range(H // 16):
            rows_v[r, pl.ds(cc * 16, 16)] = jnp.zeros((16,), jnp.float32)
        return _
    lax.fori_loop(0, K, zrow, 0)

    r0 = s * RPS
    for i in range(4):
        pltpu.sync_copy(rows_v, acc.at[pl.ds(r0 + i * K, K)])
    pltpu.sync_copy(rows_v.at[pl.ds(0, 112)], acc.at[pl.ds(r0 + 512, 112)])

    @pl.when(s == 0)
    def _():
        pltpu.sync_copy(rows_v.at[pl.ds(0, TAIL)],
                        acc.at[pl.ds(NS * RPS, TAIL)])
    plsc.subcore_barrier()

    niters = BASE_ITERS + jnp.where(wid < EXTRA, 1, 0)

    def step(t, _):
        off = (wid + t * NW) * K
        pltpu.sync_copy(src.at[pl.ds(off, K)], src_v)
        pltpu.sync_copy(dst.at[pl.ds(off, K)], dst_v)
        pltpu.async_copy(feat.at[src_v], rows_v, sem).wait()
        pltpu.sync_copy(rows_v, acc.at[dst_v], add=True)
        return _
    lax.fori_loop(0, niters, step, 0)
    plsc.subcore_barrier()

    pltpu.sync_copy(acc.at[pl.ds(r0, RPS)], part_out.at[c, pl.ds(r0, RPS)])

    @pl.when(s == 0)
    def _():
        pltpu.sync_copy(acc.at[pl.ds(NS * RPS, TAIL)],
                        part_out.at[c, pl.ds(NS * RPS, TAIL)])


def _sc_deg_body(dst, deg_out, dst_v, ones_v, dacc, sem):
    del sem
    c = lax.axis_index("c")
    s = lax.axis_index("s")
    wid = s * NC + c

    def fill(val):
        def row(r, _):
            for cc in range(H // 16):
                ones_v[r, pl.ds(cc * 16, 16)] = jnp.full((16,), val,
                                                         jnp.float32)
            return _
        lax.fori_loop(0, K, row, 0)

    fill(0.0)
    r0 = s * RPS
    for i in range(4):
        pltpu.sync_copy(ones_v, dacc.at[pl.ds(r0 + i * K, K)])
    pltpu.sync_copy(ones_v.at[pl.ds(0, 112)], dacc.at[pl.ds(r0 + 512, 112)])

    @pl.when(s == 0)
    def _():
        pltpu.sync_copy(ones_v.at[pl.ds(0, TAIL)],
                        dacc.at[pl.ds(NS * RPS, TAIL)])
    fill(1.0)
    plsc.subcore_barrier()

    niters = BASE_ITERS + jnp.where(wid < EXTRA, 1, 0)

    def step(t, _):
        off = (wid + t * NW) * K
        pltpu.sync_copy(dst.at[pl.ds(off, K)], dst_v)
        pltpu.sync_copy(ones_v, dacc.at[dst_v], add=True)
        return _
    lax.fori_loop(0, niters, step, 0)
    plsc.subcore_barrier()

    pltpu.sync_copy(dacc.at[pl.ds(r0, RPS)], deg_out.at[c, pl.ds(r0, RPS)])

    @pl.when(s == 0)
    def _():
        pltpu.sync_copy(dacc.at[pl.ds(NS * RPS, TAIL)],
                        deg_out.at[c, pl.ds(NS * RPS, TAIL)])


_SC_MESH = dict(core_axis_name="c", subcore_axis_name="s")

_sc_agg = pl.kernel(
    _sc_agg_body,
    out_type=(jax.ShapeDtypeStruct((NC, N, H), jnp.float32),),
    mesh=plsc.VectorSubcoreMesh(**_SC_MESH),
    scratch_types=(
        pltpu.VMEM((K,), jnp.int32),
        pltpu.VMEM((K,), jnp.int32),
        pltpu.VMEM((K, H), jnp.float32),
        pltpu.VMEM_SHARED((N, H), jnp.float32),
        pltpu.SemaphoreType.DMA,
    ),
)

_sc_deg = pl.kernel(
    _sc_deg_body,
    out_type=(jax.ShapeDtypeStruct((NC, N, H), jnp.float32),),
    mesh=plsc.VectorSubcoreMesh(**_SC_MESH),
    scratch_types=(
        pltpu.VMEM((K,), jnp.int32),
        pltpu.VMEM((K, H), jnp.float32),
        pltpu.VMEM_SHARED((N, H), jnp.float32),
        pltpu.SemaphoreType.DMA,
    ),
)


RB = 2000
GRID = N // RB


def _tc_layer_body(relu, x_ref, p_ref, d_ref, wn_ref, ws_ref, b_ref, o_ref):
    deg = d_ref[0, :, 0:1] + d_ref[1, :, 0:1]
    mean = (p_ref[0] + p_ref[1]) / jnp.maximum(deg, 1.0)
    out = (jnp.dot(mean, wn_ref[...], preferred_element_type=jnp.float32)
           + jnp.dot(x_ref[...], ws_ref[...],
                     preferred_element_type=jnp.float32)
           + b_ref[...])
    if relu:
        out = jnp.maximum(out, 0.0)
    o_ref[...] = out


def _tc_layer(x, parts, degp, Wn, Ws, b, relu):
    return pl.pallas_call(
        functools.partial(_tc_layer_body, relu),
        grid=(GRID,),
        in_specs=[
            pl.BlockSpec((RB, H), lambda i: (i, 0)),
            pl.BlockSpec((NC, RB, H), lambda i: (0, i, 0)),
            pl.BlockSpec((NC, RB, H), lambda i: (0, i, 0)),
            pl.BlockSpec((H, H), lambda i: (0, 0)),
            pl.BlockSpec((H, H), lambda i: (0, 0)),
            pl.BlockSpec((1, H), lambda i: (0, 0)),
        ],
        out_specs=pl.BlockSpec((RB, H), lambda i: (i, 0)),
        out_shape=jax.ShapeDtypeStruct((N, H), jnp.float32),
    )(x, parts, degp, Wn, Ws, b.reshape(1, H))


def _tc_pool_body(h_ref, batch_ref, wl_ref, bl_ref, o_ref, acc, cnt):
    i = pl.program_id(0)

    @pl.when(i == 0)
    def _():
        acc[...] = jnp.zeros_like(acc)
        cnt[...] = jnp.zeros_like(cnt)

    gids = lax.broadcasted_iota(jnp.int32, (1, G), 1)
    onehot = jnp.where(batch_ref[...] == gids, 1.0, 0.0).astype(jnp.float32)
    dn = (((0,), (0,)), ((), ()))
    acc[...] += lax.dot_general(onehot, h_ref[...], dn,
                                preferred_element_type=jnp.float32)
    cnt[...] += lax.dot_general(onehot, jnp.ones((RB, H), jnp.float32), dn,
                                preferred_element_type=jnp.float32)

    @pl.when(i == GRID - 1)
    def _():
        pooled = acc[...] / jnp.maximum(cnt[...], 1.0)
        o_ref[...] = (jnp.dot(pooled, wl_ref[...],
                              preferred_element_type=jnp.float32)
                      + bl_ref[...])


def _tc_pool(h, batch2d, Wlp, blp):
    return pl.pallas_call(
        _tc_pool_body,
        grid=(GRID,),
        in_specs=[
            pl.BlockSpec((RB, H), lambda i: (i, 0)),
            pl.BlockSpec((RB, 1), lambda i: (i, 0)),
            pl.BlockSpec((H, H), lambda i: (0, 0)),
            pl.BlockSpec((1, H), lambda i: (0, 0)),
        ],
        out_specs=pl.BlockSpec((G, H), lambda i: (0, 0)),
        out_shape=jax.ShapeDtypeStruct((G, H), jnp.float32),
        scratch_shapes=[
            pltpu.VMEM((G, H), jnp.float32),
            pltpu.VMEM((G, H), jnp.float32),
        ],
    )(h, batch2d, Wlp, blp)


def kernel(x, edge_index, batch, W1n, W1s, b1, W2n, W2s, b2, W3n, W3s, b3,
           Wl, bl):
    src = edge_index[0]
    dst = edge_index[1]

    (degp,) = _sc_deg(dst)
    (parts1,) = _sc_agg(x, src, dst)
    h1 = _tc_layer(x, parts1, degp, W1n, W1s, b1, relu=True)

    (parts2,) = _sc_agg(h1, src, dst)
    h2 = _tc_layer(h1, parts2, degp, W2n, W2s, b2, relu=True)

    (parts3,) = _sc_agg(h2, src, dst)
    h3 = _tc_layer(h2, parts3, degp, W3n, W3s, b3, relu=False)

    Wlp = jnp.zeros((H, H), jnp.float32).at[:, :Wl.shape[1]].set(Wl)
    blp = jnp.zeros((1, H), jnp.float32).at[0, :bl.shape[0]].set(bl)
    outp = _tc_pool(h3, batch.reshape(N, 1), Wlp, blp)
    out = outp[:, :Wl.shape[1]]
    return (out, h2)
